# baseline (speedup 1.0000x reference)
import jax
import jax.numpy as jnp
from jax.experimental import pallas as pl
from jax.experimental.pallas import tpu as pltpu

_HIDDEN = 128
_NSEG = 1024
_R = 3200
_G = _R // 8
_S = 64
_NL2 = 9


def _shift_lanes(v, d, fill):
    return jnp.concatenate(
        [jnp.full((1, d), fill, v.dtype), v[:, :-d]], axis=1)


def _fused_kernel(wlo_ref, whi_ref, x_ref, brow_ref, e7_ref, e0_ref, w1_ref,
                  b1_ref, w2_ref, b2_ref, out_ref, sum_s, max_s, cnt_s):
    i = pl.program_id(0)
    nb = pl.num_programs(0)

    @pl.when(i == 0)
    def _init():
        sum_s[...] = jnp.zeros_like(sum_s)
        max_s[...] = jnp.zeros_like(max_s)
        cnt_s[...] = jnp.zeros_like(cnt_s)

    x = x_ref[...]
    h = jax.lax.dot_general(x, w1_ref[...], (((1,), (1,)), ((), ())),
                            preferred_element_type=jnp.float32)
    h = jnp.maximum(h + b1_ref[...], 0.0)
    att = jax.lax.dot_general(h, w2_ref[...], (((1,), (1,)), ((), ())),
                              preferred_element_type=jnp.float32)
    att = jax.nn.sigmoid(att + b2_ref[...])
    attended = h * att

    brow = brow_ref[0]
    lane_r = jax.lax.broadcasted_iota(jnp.int32, (1, _R), 1)

    nxt = jnp.concatenate(
        [brow[:, 1:], jnp.full((1, 1), -1.0, jnp.float32)], axis=1)
    rend = ((brow != nxt) | (lane_r == _R - 1)).astype(jnp.float32)

    glast = jax.lax.dot_general(brow, e7_ref[...], (((1,), (0,)), ((), ())),
                                precision=jax.lax.Precision.HIGHEST,
                                preferred_element_type=jnp.float32)
    gfirst = jax.lax.dot_general(brow, e0_ref[...], (((1,), (0,)), ((), ())),
                                 precision=jax.lax.Precision.HIGHEST,
                                 preferred_element_type=jnp.float32)
    lane_g = jax.lax.broadcasted_iota(jnp.int32, (1, _G), 1)
    gprev_id = _shift_lanes(glast, 1, -1.0)
    gnext = jnp.concatenate(
        [gfirst[:, 1:], jnp.full((1, 1), -1.0, jnp.float32)], axis=1)
    cond = ((gprev_id == gfirst)
            & ((lane_g == _G - 1) | (gnext != gfirst))).astype(jnp.float32)

    eye = (jax.lax.broadcasted_iota(jnp.int32, (_HIDDEN, _HIDDEN), 0)
           == jax.lax.broadcasted_iota(jnp.int32, (_HIDDEN, _HIDDEN), 1)
           ).astype(jnp.float32)
    attT = jax.lax.dot_general(eye, attended, (((1,), (1,)), ((), ())),
                               preferred_element_type=jnp.float32)

    s = attT
    for d in (1, 2, 4):
        m = ((brow == _shift_lanes(brow, d, -1.0))
             & (lane_r >= d)).astype(jnp.float32)
        sh = jnp.concatenate(
            [jnp.zeros((_HIDDEN, d), jnp.float32), s[:, :-d]], axis=1)
        s = jnp.maximum(s, sh * m)
    scannedT = s

    gsumT = jax.lax.dot_general(scannedT, e7_ref[...], (((1,), (0,)), ((), ())),
                                preferred_element_type=jnp.float32)

    t = gsumT
    d = 1
    for _ in range(_NL2):
        m = ((glast == _shift_lanes(glast, d, -1.0))
             & (lane_g >= d)).astype(jnp.float32)
        sh = jnp.concatenate(
            [jnp.zeros((_HIDDEN, d), jnp.float32), t[:, :-d]], axis=1)
        t = jnp.maximum(t, sh * m)
        d *= 2
    gprevT = jnp.concatenate(
        [jnp.zeros((_HIDDEN, 1), jnp.float32), t[:, :-1]], axis=1)

    def _window(w, carry):
        base = w * _S
        basef = base.astype(jnp.float32)
        iota_r = jax.lax.broadcasted_iota(
            jnp.int32, (_S, _R), 0).astype(jnp.float32)
        oh = (brow - basef == iota_r).astype(jnp.float32)
        sums_u = jax.lax.dot_general(oh, attended, (((1,), (0,)), ((), ())),
                                     preferred_element_type=jnp.float32)
        cnts_u = jnp.sum(oh, axis=1, keepdims=True)
        sel1 = oh * rend
        max1 = jax.lax.dot_general(sel1, scannedT, (((1,), (1,)), ((), ())),
                                   preferred_element_type=jnp.float32)
        iota_g = jax.lax.broadcasted_iota(
            jnp.int32, (_S, _G), 0).astype(jnp.float32)
        ohg = (gfirst - basef == iota_g).astype(jnp.float32) * cond
        max2 = jax.lax.dot_general(ohg, gprevT, (((1,), (1,)), ((), ())),
                                   preferred_element_type=jnp.float32)
        maxs_u = jnp.maximum(max1, max2)
        sum_s[pl.ds(base, _S), :] += sums_u
        cnt_s[pl.ds(base, _S), :] += cnts_u
        max_s[pl.ds(base, _S), :] = jnp.maximum(max_s[pl.ds(base, _S), :],
                                                maxs_u)
        return carry

    jax.lax.fori_loop(wlo_ref[i], whi_ref[i] + 1, _window, 0)

    @pl.when(i == nb - 1)
    def _finish():
        cnt = cnt_s[...]
        out_ref[:, :_HIDDEN] = max_s[...]
        out_ref[:, _HIDDEN:] = sum_s[...] / jnp.maximum(cnt, 1.0)


@jax.jit
def kernel(x, batch, W1, b1, W2, b2):
    n = x.shape[0]
    assert n % _R == 0
    nb = n // _R
    batch = batch.astype(jnp.int32)
    wlo = (batch[::_R] // _S).astype(jnp.int32)
    whi = (batch[_R - 1::_R] // _S).astype(jnp.int32)
    brow = batch.astype(jnp.float32).reshape(nb, 1, _R)

    r_ids = jnp.arange(_R, dtype=jnp.int32)[:, None]
    g_ids = jnp.arange(_G, dtype=jnp.int32)[None, :]
    e7 = (r_ids == g_ids * 8 + 7).astype(jnp.float32)
    e0 = (r_ids == g_ids * 8).astype(jnp.float32)

    b1r = b1.reshape(1, _HIDDEN)
    b2r = b2.reshape(1, _HIDDEN)

    grid_spec = pltpu.PrefetchScalarGridSpec(
        num_scalar_prefetch=2,
        grid=(nb,),
        in_specs=[
            pl.BlockSpec((_R, _HIDDEN), lambda i, *_: (i, 0)),
            pl.BlockSpec((1, 1, _R), lambda i, *_: (i, 0, 0)),
            pl.BlockSpec((_R, _G), lambda i, *_: (0, 0)),
            pl.BlockSpec((_R, _G), lambda i, *_: (0, 0)),
            pl.BlockSpec((_HIDDEN, _HIDDEN), lambda i, *_: (0, 0)),
            pl.BlockSpec((1, _HIDDEN), lambda i, *_: (0, 0)),
            pl.BlockSpec((_HIDDEN, _HIDDEN), lambda i, *_: (0, 0)),
            pl.BlockSpec((1, _HIDDEN), lambda i, *_: (0, 0)),
        ],
        out_specs=pl.BlockSpec((_NSEG, 2 * _HIDDEN), lambda i, *_: (0, 0)),
        scratch_shapes=[
            pltpu.VMEM((_NSEG, _HIDDEN), jnp.float32),
            pltpu.VMEM((_NSEG, _HIDDEN), jnp.float32),
            pltpu.VMEM((_NSEG, 1), jnp.float32),
        ],
    )
    out = pl.pallas_call(
        _fused_kernel,
        grid_spec=grid_spec,
        out_shape=jax.ShapeDtypeStruct((_NSEG, 2 * _HIDDEN), jnp.float32),
        compiler_params=pltpu.CompilerParams(
            dimension_semantics=("arbitrary",)),
    )(wlo, whi, x, brow, e7, e0, W1, b1r, W2, b2r)
    return out


# speedup vs baseline: 2.2454x; 2.2454x over previous
import jax
import jax.numpy as jnp
from jax.experimental import pallas as pl
from jax.experimental.pallas import tpu as pltpu

_HIDDEN = 128
_NSEG = 1024
_R = 3200
_G = _R // 8
_S = 64
_NL2 = 9


def _shift_lanes(v, d, fill):
    return jnp.concatenate(
        [jnp.full((1, d), fill, v.dtype), v[:, :-d]], axis=1)


def _fused_kernel(wlo_ref, whi_ref, x_ref, brow_ref, grow_ref, e7_ref,
                  w1_ref, b1_ref, w2_ref, b2_ref, out_ref, sum_s, max_s,
                  cnt_s):
    i = pl.program_id(0)
    nb = pl.num_programs(0)

    @pl.when(i == 0)
    def _init():
        sum_s[...] = jnp.zeros_like(sum_s)
        max_s[...] = jnp.zeros_like(max_s)
        cnt_s[...] = jnp.zeros_like(cnt_s)

    x = x_ref[...]
    h = jax.lax.dot_general(x, w1_ref[...], (((1,), (1,)), ((), ())),
                            preferred_element_type=jnp.float32)
    h = jnp.maximum(h + b1_ref[...], 0.0)
    att = jax.lax.dot_general(h, w2_ref[...], (((1,), (1,)), ((), ())),
                              preferred_element_type=jnp.float32)
    att = jax.nn.sigmoid(att + b2_ref[...])
    attended = h * att

    brow = brow_ref[0]
    lane_r = jax.lax.broadcasted_iota(jnp.int32, (1, _R), 1)

    nxt = jnp.concatenate(
        [brow[:, 1:], jnp.full((1, 1), -1.0, jnp.float32)], axis=1)
    rend = ((brow != nxt) | (lane_r == _R - 1)).astype(jnp.float32)

    grow = grow_ref[0]
    glast = grow[:, 0:_G]
    gfirst = grow[:, _G:2 * _G]
    lane_g = jax.lax.broadcasted_iota(jnp.int32, (1, _G), 1)
    gprev_id = _shift_lanes(glast, 1, -1.0)
    gnext = jnp.concatenate(
        [gfirst[:, 1:], jnp.full((1, 1), -1.0, jnp.float32)], axis=1)
    cond = ((gprev_id == gfirst)
            & ((lane_g == _G - 1) | (gnext != gfirst))).astype(jnp.float32)

    eye = (jax.lax.broadcasted_iota(jnp.int32, (_HIDDEN, _HIDDEN), 0)
           == jax.lax.broadcasted_iota(jnp.int32, (_HIDDEN, _HIDDEN), 1)
           ).astype(jnp.float32)
    attT = jax.lax.dot_general(eye, attended, (((1,), (1,)), ((), ())),
                               preferred_element_type=jnp.float32)

    s = attT
    for d in (1, 2, 4):
        m = ((brow == _shift_lanes(brow, d, -1.0))
             & (lane_r >= d)).astype(jnp.float32)
        sh = jnp.concatenate(
            [jnp.zeros((_HIDDEN, d), jnp.float32), s[:, :-d]], axis=1)
        s = jnp.maximum(s, sh * m)
    scannedT = s

    gsumT = jax.lax.dot_general(scannedT, e7_ref[...], (((1,), (0,)), ((), ())),
                                preferred_element_type=jnp.float32)

    t = gsumT
    d = 1
    for _ in range(_NL2):
        m = ((glast == _shift_lanes(glast, d, -1.0))
             & (lane_g >= d)).astype(jnp.float32)
        sh = jnp.concatenate(
            [jnp.zeros((_HIDDEN, d), jnp.float32), t[:, :-d]], axis=1)
        t = jnp.maximum(t, sh * m)
        d *= 2
    gprevT = jnp.concatenate(
        [jnp.zeros((_HIDDEN, 1), jnp.float32), t[:, :-1]], axis=1)

    def _window(w, carry):
        base = w * _S
        basef = base.astype(jnp.float32)
        iota_r = jax.lax.broadcasted_iota(
            jnp.int32, (_S, _R), 0).astype(jnp.float32)
        oh = (brow - basef == iota_r).astype(jnp.float32)
        sums_u = jax.lax.dot_general(oh, attended, (((1,), (0,)), ((), ())),
                                     preferred_element_type=jnp.float32)
        cnts_u = jnp.sum(oh, axis=1, keepdims=True)
        sel1 = oh * rend
        max1 = jax.lax.dot_general(sel1, scannedT, (((1,), (1,)), ((), ())),
                                   preferred_element_type=jnp.float32)
        iota_g = jax.lax.broadcasted_iota(
            jnp.int32, (_S, _G), 0).astype(jnp.float32)
        ohg = (gfirst - basef == iota_g).astype(jnp.float32) * cond
        max2 = jax.lax.dot_general(ohg, gprevT, (((1,), (1,)), ((), ())),
                                   preferred_element_type=jnp.float32)
        maxs_u = jnp.maximum(max1, max2)
        sum_s[pl.ds(base, _S), :] += sums_u
        cnt_s[pl.ds(base, _S), :] += cnts_u
        max_s[pl.ds(base, _S), :] = jnp.maximum(max_s[pl.ds(base, _S), :],
                                                maxs_u)
        return carry

    jax.lax.fori_loop(wlo_ref[i], whi_ref[i] + 1, _window, 0)

    @pl.when(i == nb - 1)
    def _finish():
        cnt = cnt_s[...]
        out_ref[:, :_HIDDEN] = max_s[...]
        out_ref[:, _HIDDEN:] = sum_s[...] / jnp.maximum(cnt, 1.0)


@jax.jit
def kernel(x, batch, W1, b1, W2, b2):
    n = x.shape[0]
    assert n % _R == 0
    nb = n // _R
    batch = batch.astype(jnp.int32)
    wlo = (batch[::_R] // _S).astype(jnp.int32)
    whi = (batch[_R - 1::_R] // _S).astype(jnp.int32)
    brow = batch.astype(jnp.float32).reshape(nb, 1, _R)

    r_ids = jnp.arange(_R, dtype=jnp.int32)[:, None]
    g_ids = jnp.arange(_G, dtype=jnp.int32)[None, :]
    e7 = (r_ids == g_ids * 8 + 7).astype(jnp.float32)
    b8 = batch.reshape(-1, 8)
    grow = (jnp.concatenate(
        [b8[:, 7].reshape(nb, _G), b8[:, 0].reshape(nb, _G)], axis=1)
        .astype(jnp.float32).reshape(nb, 1, 2 * _G))

    b1r = b1.reshape(1, _HIDDEN)
    b2r = b2.reshape(1, _HIDDEN)

    grid_spec = pltpu.PrefetchScalarGridSpec(
        num_scalar_prefetch=2,
        grid=(nb,),
        in_specs=[
            pl.BlockSpec((_R, _HIDDEN), lambda i, *_: (i, 0)),
            pl.BlockSpec((1, 1, _R), lambda i, *_: (i, 0, 0)),
            pl.BlockSpec((1, 1, 2 * _G), lambda i, *_: (i, 0, 0)),
            pl.BlockSpec((_R, _G), lambda i, *_: (0, 0)),
            pl.BlockSpec((_HIDDEN, _HIDDEN), lambda i, *_: (0, 0)),
            pl.BlockSpec((1, _HIDDEN), lambda i, *_: (0, 0)),
            pl.BlockSpec((_HIDDEN, _HIDDEN), lambda i, *_: (0, 0)),
            pl.BlockSpec((1, _HIDDEN), lambda i, *_: (0, 0)),
        ],
        out_specs=pl.BlockSpec((_NSEG, 2 * _HIDDEN), lambda i, *_: (0, 0)),
        scratch_shapes=[
            pltpu.VMEM((_NSEG, _HIDDEN), jnp.float32),
            pltpu.VMEM((_NSEG, _HIDDEN), jnp.float32),
            pltpu.VMEM((_NSEG, 1), jnp.float32),
        ],
    )
    out = pl.pallas_call(
        _fused_kernel,
        grid_spec=grid_spec,
        out_shape=jax.ShapeDtypeStruct((_NSEG, 2 * _HIDDEN), jnp.float32),
        compiler_params=pltpu.CompilerParams(
            dimension_semantics=("arbitrary",)),
    )(wlo, whi, x, brow, grow, e7, W1, b1r, W2, b2r)
    return out
